# Initial kernel scaffold; baseline (speedup 1.0000x reference)
import jax
import jax.numpy as jnp
from jax.experimental import pallas as pl


def kernel(feats, edge_index, W1, b1, W2, b2, Wl1, bl1, Wl2, bl2, Wl3, bl3, Wl4, bl4, Wl5, bl5):
    raise NotImplementedError("write your pallas kernel here")


# speedup vs baseline: 2.6428x; 2.6428x over previous
import functools

import jax
import jax.numpy as jnp
from jax import lax
from jax.experimental import pallas as pl
from jax.experimental.pallas import tpu as pltpu
from jax.experimental.pallas import tpu_sc as plsc

N = 10000
E = 320000
D = 128

NC = 2
NS = 16
NW = NC * NS

CH = 128
EPW = 10240
E_PAD = EPW * NW
N_PAD = 10112
RZ = N_PAD // NS


def _sc_agg_body(x_hbm, src_hbm, dst_hbm, zeros_hbm, out_hbm,
                 src_v, dst_v, rows_v, agg_sh, sem):
    c = lax.axis_index("c")
    s = lax.axis_index("s")
    wid = c * NS + s

    pltpu.sync_copy(zeros_hbm.at[pl.ds(s * RZ, RZ)], agg_sh.at[pl.ds(s * RZ, RZ)])
    plsc.subcore_barrier()

    ebase = wid * EPW

    def body(i, carry):
        off = ebase + i * CH
        pltpu.sync_copy(src_hbm.at[pl.ds(off, CH)], src_v)
        pltpu.sync_copy(dst_hbm.at[pl.ds(off, CH)], dst_v)
        pltpu.async_copy(x_hbm.at[src_v], rows_v, sem).wait()
        pltpu.sync_copy(rows_v, agg_sh.at[dst_v], add=True)
        return carry

    lax.fori_loop(0, EPW // CH, body, 0)
    plsc.subcore_barrier()

    pltpu.sync_copy(agg_sh.at[pl.ds(s * RZ, RZ)], out_hbm.at[c, pl.ds(s * RZ, RZ)])


_sc_agg = functools.partial(
    pl.kernel,
    out_type=jax.ShapeDtypeStruct((NC, N_PAD, D), jnp.float32),
    mesh=plsc.VectorSubcoreMesh(core_axis_name="c", subcore_axis_name="s"),
    scratch_types=[
        pltpu.VMEM((CH,), jnp.int32),
        pltpu.VMEM((CH,), jnp.int32),
        pltpu.VMEM((CH, D), jnp.float32),
        pltpu.VMEM_SHARED((N_PAD, D), jnp.float32),
        pltpu.SemaphoreType.DMA,
    ],
)(_sc_agg_body)


def _leaky(x):
    return jnp.where(x > 0, x, 0.01 * x)


BR = 1000


def _tc1_body(x_ref, p_ref, w_ref, b_ref, o_ref):
    h = x_ref[...] + p_ref[0] + p_ref[1]
    y = jnp.dot(h, w_ref[...], preferred_element_type=jnp.float32) + b_ref[...]
    o_ref[...] = _leaky(y)


def _tc2_body(x_ref, p_ref, w2, b2, wl1, bl1, wl2, bl2, wl3, bl3, wl4, bl4,
              wl5, bl5, o_ref):
    h = x_ref[...] + p_ref[0] + p_ref[1]
    h = _leaky(jnp.dot(h, w2[...], preferred_element_type=jnp.float32) + b2[...])
    h = _leaky(jnp.dot(h, wl1[...], preferred_element_type=jnp.float32) + bl1[...])
    h = _leaky(jnp.dot(h, wl2[...], preferred_element_type=jnp.float32) + bl2[...])
    h = _leaky(jnp.dot(h, wl3[...], preferred_element_type=jnp.float32) + bl3[...])
    h = _leaky(jnp.dot(h, wl4[...], preferred_element_type=jnp.float32) + bl4[...])
    o_ref[...] = jnp.dot(h, wl5[...], preferred_element_type=jnp.float32) + bl5[...]


def _row_spec():
    return pl.BlockSpec((BR, D), lambda i: (i, 0))


def _agg_spec():
    return pl.BlockSpec((NC, BR, D), lambda i: (0, i, 0))


def _w_spec():
    return pl.BlockSpec((D, D), lambda i: (0, 0))


def _b_spec():
    return pl.BlockSpec((1, D), lambda i: (0, 0))


def _tc1(x, agg, w, b):
    return pl.pallas_call(
        _tc1_body,
        grid=(N // BR,),
        in_specs=[_row_spec(), _agg_spec(), _w_spec(), _b_spec()],
        out_specs=_row_spec(),
        out_shape=jax.ShapeDtypeStruct((N, D), jnp.float32),
    )(x, agg, w, b)


def _tc2(x, agg, w2, b2, wl1, bl1, wl2, bl2, wl3, bl3, wl4, bl4, wl5, bl5):
    wb_specs = [_w_spec(), _b_spec()] * 6
    return pl.pallas_call(
        _tc2_body,
        grid=(N // BR,),
        in_specs=[_row_spec(), _agg_spec()] + wb_specs,
        out_specs=_row_spec(),
        out_shape=jax.ShapeDtypeStruct((N, D), jnp.float32),
    )(x, agg, w2, b2, wl1, bl1, wl2, bl2, wl3, bl3, wl4, bl4, wl5, bl5)


def kernel(feats, edge_index, W1, b1, W2, b2, Wl1, bl1, Wl2, bl2, Wl3, bl3,
           Wl4, bl4, Wl5, bl5):
    src = edge_index[0].astype(jnp.int32)
    dst = edge_index[1].astype(jnp.int32)
    npad = E_PAD - E
    src = jnp.concatenate([src, jnp.zeros((npad,), jnp.int32)])
    dst = jnp.concatenate([dst, jnp.full((npad,), N, jnp.int32)])
    zeros = jnp.zeros((N_PAD, D), jnp.float32)

    b1r = b1.reshape(1, D)
    b2r = b2.reshape(1, D)
    bl1r = bl1.reshape(1, D)
    bl2r = bl2.reshape(1, D)
    bl3r = bl3.reshape(1, D)
    bl4r = bl4.reshape(1, D)
    bl5r = bl5.reshape(1, D)

    agg1 = _sc_agg(feats, src, dst, zeros)
    h1 = _tc1(feats, agg1, W1, b1r)
    agg2 = _sc_agg(h1, src, dst, zeros)
    out = _tc2(h1, agg2, W2, b2r, Wl1, bl1r, Wl2, bl2r, Wl3, bl3r, Wl4, bl4r,
               Wl5, bl5r)
    return out


# speedup vs baseline: 3.3025x; 1.2496x over previous
import functools

import jax
import jax.numpy as jnp
from jax import lax
from jax.experimental import pallas as pl
from jax.experimental.pallas import tpu as pltpu
from jax.experimental.pallas import tpu_sc as plsc

N = 10000
E = 320000
D = 128

NC = 2
NS = 16
NW = NC * NS

CH = 128
EPW = 10240
E_PAD = EPW * NW
N_PAD = 10112
RZ = N_PAD // NS


CPW = EPW // CH
NBUF = 2
HALF = CPW // 2


def _sc_agg_body(x_hbm, src_hbm, dst_hbm, zeros_hbm, out_hbm,
                 src_sl, dst_sl, rows, sems, agg_sh):
    c = lax.axis_index("c")
    s = lax.axis_index("s")
    wid = c * NS + s

    pltpu.sync_copy(zeros_hbm.at[pl.ds(s * RZ, RZ)], agg_sh.at[pl.ds(s * RZ, RZ)])
    plsc.subcore_barrier()

    def start_gather(j, b):
        pltpu.make_async_copy(x_hbm.at[src_sl.at[j]], rows[b], sems[b]).start()

    def wait_gather(j, b):
        pltpu.make_async_copy(x_hbm.at[src_sl.at[j]], rows[b], sems[b]).wait()

    for p in range(2):
        pltpu.sync_copy(src_hbm.at[wid, pl.ds(p * HALF, HALF)], src_sl)
        pltpu.sync_copy(dst_hbm.at[wid, pl.ds(p * HALF, HALF)], dst_sl)

        for b in range(NBUF):
            start_gather(b, b)

        def body(i, carry):
            j0 = i * NBUF
            for b in range(NBUF):
                j = j0 + b
                wait_gather(j, b)
                pltpu.sync_copy(rows[b], agg_sh.at[dst_sl.at[j]], add=True)

                @pl.when(j + NBUF < HALF)
                def _():
                    start_gather(j + NBUF, b)

            return carry

        lax.fori_loop(0, HALF // NBUF, body, 0)

    plsc.subcore_barrier()

    pltpu.sync_copy(agg_sh.at[pl.ds(s * RZ, RZ)], out_hbm.at[c, pl.ds(s * RZ, RZ)])


_sc_agg = functools.partial(
    pl.kernel,
    out_type=jax.ShapeDtypeStruct((NC, N_PAD, D), jnp.float32),
    mesh=plsc.VectorSubcoreMesh(core_axis_name="c", subcore_axis_name="s"),
    scratch_types=[
        pltpu.VMEM((HALF, CH), jnp.int32),
        pltpu.VMEM((HALF, CH), jnp.int32),
        [pltpu.VMEM((CH, D), jnp.float32)] * NBUF,
        [pltpu.SemaphoreType.DMA] * NBUF,
        pltpu.VMEM_SHARED((N_PAD, D), jnp.float32),
    ],
)(_sc_agg_body)


def _leaky(x):
    return jnp.where(x > 0, x, 0.01 * x)


BR = 1000


def _tc1_body(x_ref, p_ref, w_ref, b_ref, o_ref):
    h = x_ref[...] + p_ref[0] + p_ref[1]
    y = jnp.dot(h, w_ref[...], preferred_element_type=jnp.float32) + b_ref[...]
    o_ref[...] = _leaky(y)


def _tc2_body(x_ref, p_ref, w2, b2, wl1, bl1, wl2, bl2, wl3, bl3, wl4, bl4,
              wl5, bl5, o_ref):
    h = x_ref[...] + p_ref[0] + p_ref[1]
    h = _leaky(jnp.dot(h, w2[...], preferred_element_type=jnp.float32) + b2[...])
    h = _leaky(jnp.dot(h, wl1[...], preferred_element_type=jnp.float32) + bl1[...])
    h = _leaky(jnp.dot(h, wl2[...], preferred_element_type=jnp.float32) + bl2[...])
    h = _leaky(jnp.dot(h, wl3[...], preferred_element_type=jnp.float32) + bl3[...])
    h = _leaky(jnp.dot(h, wl4[...], preferred_element_type=jnp.float32) + bl4[...])
    o_ref[...] = jnp.dot(h, wl5[...], preferred_element_type=jnp.float32) + bl5[...]


def _row_spec():
    return pl.BlockSpec((BR, D), lambda i: (i, 0))


def _agg_spec():
    return pl.BlockSpec((NC, BR, D), lambda i: (0, i, 0))


def _w_spec():
    return pl.BlockSpec((D, D), lambda i: (0, 0))


def _b_spec():
    return pl.BlockSpec((1, D), lambda i: (0, 0))


def _tc1(x, agg, w, b):
    return pl.pallas_call(
        _tc1_body,
        grid=(N // BR,),
        in_specs=[_row_spec(), _agg_spec(), _w_spec(), _b_spec()],
        out_specs=_row_spec(),
        out_shape=jax.ShapeDtypeStruct((N, D), jnp.float32),
    )(x, agg, w, b)


def _tc2(x, agg, w2, b2, wl1, bl1, wl2, bl2, wl3, bl3, wl4, bl4, wl5, bl5):
    wb_specs = [_w_spec(), _b_spec()] * 6
    return pl.pallas_call(
        _tc2_body,
        grid=(N // BR,),
        in_specs=[_row_spec(), _agg_spec()] + wb_specs,
        out_specs=_row_spec(),
        out_shape=jax.ShapeDtypeStruct((N, D), jnp.float32),
    )(x, agg, w2, b2, wl1, bl1, wl2, bl2, wl3, bl3, wl4, bl4, wl5, bl5)


def kernel(feats, edge_index, W1, b1, W2, b2, Wl1, bl1, Wl2, bl2, Wl3, bl3,
           Wl4, bl4, Wl5, bl5):
    src = edge_index[0].astype(jnp.int32)
    dst = edge_index[1].astype(jnp.int32)
    npad = E_PAD - E
    src = jnp.concatenate([src, jnp.zeros((npad,), jnp.int32)])
    dst = jnp.concatenate([dst, jnp.full((npad,), N, jnp.int32)])
    src = src.reshape(NW, CPW, CH)
    dst = dst.reshape(NW, CPW, CH)
    zeros = jnp.zeros((N_PAD, D), jnp.float32)

    b1r = b1.reshape(1, D)
    b2r = b2.reshape(1, D)
    bl1r = bl1.reshape(1, D)
    bl2r = bl2.reshape(1, D)
    bl3r = bl3.reshape(1, D)
    bl4r = bl4.reshape(1, D)
    bl5r = bl5.reshape(1, D)

    agg1 = _sc_agg(feats, src, dst, zeros)
    h1 = _tc1(feats, agg1, W1, b1r)
    agg2 = _sc_agg(h1, src, dst, zeros)
    out = _tc2(h1, agg2, W2, b2r, Wl1, bl1r, Wl2, bl2r, Wl3, bl3r, Wl4, bl4r,
               Wl5, bl5r)
    return out
